# baseline (speedup 1.0000x reference)
import functools
import math

import jax
import jax.numpy as jnp
from jax import lax
from jax.experimental import pallas as pl
from jax.experimental.pallas import tpu as pltpu
from jax.experimental.pallas import tpu_sc as plsc

N_NODES = 10000
DIM = 256
NH = 8
CPH = 32
E_EDGES = 160000

ROW = 144
DROW = 16
ACC_ROWS = 10016
CHUNK = 64
N_TILES = 16
EPB_CHUNKS = 2 * (-(-E_EDGES // (2 * N_TILES * CHUNK)))
EPB = EPB_CHUNKS * CHUNK
EPAD = EPB * N_TILES
EGLUE = EPAD + CHUNK
BLK = 1000
GRID = N_NODES // BLK
ZSTRIPE = ACC_ROWS // N_TILES


def _a_body(x_ref, w_ref, as_ref, ad_ref, ts0_ref, ts1_ref, td0_ref, td1_ref, h_ref):
    xb = x_ref[...]
    h = jnp.dot(xb, w_ref[...], preferred_element_type=jnp.float32)
    al_s = jnp.dot(h, as_ref[...], preferred_element_type=jnp.float32)
    al_d = jnp.dot(h, ad_ref[...], preferred_element_type=jnp.float32)
    e0 = al_s + al_d
    m_init = jnp.where(e0 > 0.0, e0, 0.2 * e0)
    h_ref[...] = h
    z12 = jnp.zeros((BLK, 12), jnp.float32)
    z8 = jnp.zeros((BLK, 8), jnp.float32)
    ts0_ref[...] = jnp.concatenate([al_s[:, 0:4], z12, h[:, 0:128]], axis=1)
    ts1_ref[...] = jnp.concatenate([al_s[:, 4:8], z12, h[:, 128:256]], axis=1)
    td0_ref[...] = jnp.concatenate([al_d[:, 0:4], m_init[:, 0:4], z8], axis=1)
    td1_ref[...] = jnp.concatenate([al_d[:, 4:8], m_init[:, 4:8], z8], axis=1)


def _project_tables(x, w, a_sel_s, a_sel_d):
    return pl.pallas_call(
        _a_body,
        grid=(GRID,),
        in_specs=[
            pl.BlockSpec((BLK, DIM), lambda i: (i, 0)),
            pl.BlockSpec((DIM, DIM), lambda i: (0, 0)),
            pl.BlockSpec((DIM, NH), lambda i: (0, 0)),
            pl.BlockSpec((DIM, NH), lambda i: (0, 0)),
        ],
        out_specs=[
            pl.BlockSpec((BLK, ROW), lambda i: (i, 0)),
            pl.BlockSpec((BLK, ROW), lambda i: (i, 0)),
            pl.BlockSpec((BLK, DROW), lambda i: (i, 0)),
            pl.BlockSpec((BLK, DROW), lambda i: (i, 0)),
            pl.BlockSpec((BLK, DIM), lambda i: (i, 0)),
        ],
        out_shape=[
            jax.ShapeDtypeStruct((N_NODES, ROW), jnp.float32),
            jax.ShapeDtypeStruct((N_NODES, ROW), jnp.float32),
            jax.ShapeDtypeStruct((N_NODES, DROW), jnp.float32),
            jax.ShapeDtypeStruct((N_NODES, DROW), jnp.float32),
            jax.ShapeDtypeStruct((N_NODES, DIM), jnp.float32),
        ],
    )(x, w, a_sel_s, a_sel_d)


def _b_body(ts_hbm, td_hbm, gsrc_hbm, gdst_hbm, dsts_hbm, zero_hbm, acc_hbm,
            sidx0, sidx1, didx0, didx1, scat0, scat1,
            srows0, srows1, drows0, drows1, orows0, orows1, acc_sh,
            sem_s0, sem_s1, sem_d0, sem_d1):
    cid = lax.axis_index("c")
    sid = lax.axis_index("s")
    bufs = [
        (sidx0, didx0, scat0, srows0, drows0, orows0, sem_s0, sem_d0),
        (sidx1, didx1, scat1, srows1, drows1, orows1, sem_s1, sem_d1),
    ]

    pltpu.sync_copy(zero_hbm.at[pl.ds(sid * ZSTRIPE, ZSTRIPE)],
                    acc_sh.at[pl.ds(sid * ZSTRIPE, ZSTRIPE)])

    for _, _, _, _, _, orows, _, _ in bufs:
        @plsc.parallel_loop(0, CHUNK // 16)
        def zrow(g, orows=orows):
            rows = lax.iota(jnp.int32, 16) + g * 16
            for j in range(4, 16):
                plsc.store_scatter(orows, [rows, jnp.full((16,), j, jnp.int32)],
                                   jnp.zeros((16,), jnp.float32))
    plsc.subcore_barrier()

    def prefetch(ci, b):
        sidx, didx, scat, srows, drows, _, sem_s, sem_d = b
        base = sid * EPB + ci * CHUNK
        pltpu.sync_copy(gsrc_hbm.at[cid, pl.ds(base, CHUNK)], sidx)
        pltpu.sync_copy(gdst_hbm.at[cid, pl.ds(base, CHUNK)], didx)
        pltpu.sync_copy(dsts_hbm.at[pl.ds(base, CHUNK)], scat)
        pltpu.async_copy(ts_hbm.at[sidx], srows, sem_s)
        pltpu.async_copy(td_hbm.at[didx], drows, sem_d)

    def gwait(b):
        sidx, didx, _, srows, drows, _, sem_s, sem_d = b
        pltpu.make_async_copy(ts_hbm.at[sidx], srows, sem_s).wait()
        pltpu.make_async_copy(td_hbm.at[didx], drows, sem_d).wait()

    def scatter(b):
        _, _, scat, _, _, orows, _, _ = b
        pltpu.sync_copy(orows, acc_sh.at[scat], add=True)

    def compute(b):
        _, _, _, srows, drows, orows, _, _ = b

        @plsc.parallel_loop(0, CHUNK // 16)
        def grp(g):
            rows = lax.iota(jnp.int32, 16) + g * 16
            tks = []
            for k in range(4):
                colk = jnp.full((16,), k, jnp.int32)
                a_s = plsc.load_gather(srows, [rows, colk])
                a_d = plsc.load_gather(drows, [rows, colk])
                m = plsc.load_gather(drows, [rows, jnp.full((16,), 4 + k, jnp.int32)])
                e = a_s + a_d
                e = jnp.where(e > 0.0, e, 0.2 * e)
                tk = jnp.exp(e - m)
                tks.append(tk)
                plsc.store_scatter(orows, [rows, colk], tk)
            for cc in range(128):
                col = jnp.full((16,), 16 + cc, jnp.int32)
                hv = plsc.load_gather(srows, [rows, col])
                plsc.store_scatter(orows, [rows, col], hv * tks[cc // 32])

    prefetch(0, bufs[0])

    def chunk_pair(kk, carry):
        ci = kk * 2
        prefetch(ci + 1, bufs[1])
        gwait(bufs[0])
        compute(bufs[0])
        scatter(bufs[0])
        prefetch(ci + 2, bufs[0])
        gwait(bufs[1])
        compute(bufs[1])
        scatter(bufs[1])
        return carry

    lax.fori_loop(0, EPB_CHUNKS // 2, chunk_pair, 0, unroll=False)
    gwait(bufs[0])
    plsc.subcore_barrier()
    pltpu.sync_copy(acc_sh.at[pl.ds(sid * ZSTRIPE, ZSTRIPE)],
                    acc_hbm.at[cid, pl.ds(sid * ZSTRIPE, ZSTRIPE)])


def _edge_aggregate(table_src, table_dst, gsrc, gdst, dsts, zeros_init):
    mesh = plsc.VectorSubcoreMesh(core_axis_name="c", subcore_axis_name="s",
                                  num_cores=2, num_subcores=N_TILES)
    return pl.kernel(
        _b_body,
        out_type=jax.ShapeDtypeStruct((2, ACC_ROWS, ROW), jnp.float32),
        mesh=mesh,
        scratch_types=(
            [pltpu.VMEM((CHUNK,), jnp.int32)] * 6
            + [pltpu.VMEM((CHUNK, ROW), jnp.float32)] * 2
            + [pltpu.VMEM((CHUNK, DROW), jnp.float32)] * 2
            + [pltpu.VMEM((CHUNK, ROW), jnp.float32)] * 2
            + [pltpu.VMEM_SHARED((ACC_ROWS, ROW), jnp.float32)]
            + [pltpu.SemaphoreType.DMA] * 4
        ),
        compiler_params=pltpu.CompilerParams(use_tc_tiling_on_sc=False,
                                             needs_layout_passes=False),
    )(table_src, table_dst, gsrc, gdst, dsts, zeros_init)


def _c_body(x_ref, h_ref, a0_ref, a1_ref, gb_ref, lg_ref, lb_ref,
            w1_ref, b1_ref, w2_ref, b2_ref, o_ref):
    xb = x_ref[...]
    hb = h_ref[...]
    a0 = a0_ref[...]
    a1 = a1_ref[...]
    msg = jnp.concatenate([a0[:, 16:ROW], a1[:, 16:ROW]], axis=1)
    den = 1.0 + jnp.concatenate([a0[:, 0:4], a1[:, 0:4]], axis=1)
    rec = 1.0 / den
    head_of = lax.broadcasted_iota(jnp.int32, (NH, DIM), 1) // CPH
    hid = lax.broadcasted_iota(jnp.int32, (NH, DIM), 0)
    sel = (head_of == hid).astype(jnp.float32)
    recb = jnp.dot(rec, sel, preferred_element_type=jnp.float32)
    gat = (hb + msg) * recb + gb_ref[...]

    lg = lg_ref[...]
    lb = lb_ref[...]

    def ln(v):
        mu = jnp.mean(v, axis=-1, keepdims=True)
        var = jnp.mean((v - mu) ** 2, axis=-1, keepdims=True)
        return (v - mu) * lax.rsqrt(var + 1e-5) * lg + lb

    h1 = ln(xb + gat)
    aa = jnp.dot(h1, w1_ref[...], preferred_element_type=jnp.float32) + b1_ref[...]
    gg = 0.5 * aa * (1.0 + lax.erf(aa * (1.0 / math.sqrt(2.0))))
    ff = jnp.dot(gg, w2_ref[...], preferred_element_type=jnp.float32) + b2_ref[...]
    o_ref[...] = ln(h1 + ff)


def _ffn_block(x, h, acc, gat_bias, ln_g, ln_b, w1, b1, w2, b2):
    full = lambda shape: pl.BlockSpec(shape, lambda i: tuple(0 for _ in shape))
    return pl.pallas_call(
        _c_body,
        grid=(GRID,),
        in_specs=[
            pl.BlockSpec((BLK, DIM), lambda i: (i, 0)),
            pl.BlockSpec((BLK, DIM), lambda i: (i, 0)),
            pl.BlockSpec((None, BLK, ROW), lambda i: (0, i, 0)),
            pl.BlockSpec((None, BLK, ROW), lambda i: (1, i, 0)),
            full((1, DIM)),
            full((1, DIM)),
            full((1, DIM)),
            full((DIM, 4 * DIM)),
            full((1, 4 * DIM)),
            full((4 * DIM, DIM)),
            full((1, DIM)),
        ],
        out_specs=pl.BlockSpec((BLK, DIM), lambda i: (i, 0)),
        out_shape=jax.ShapeDtypeStruct((N_NODES, DIM), jnp.float32),
    )(x, h, acc, acc, gat_bias, ln_g, ln_b, w1, b1, w2, b2)


def kernel(x, edge_index, W, att_src, att_dst, gat_bias, ln_g, ln_b, W1, b1, W2, b2):
    src = edge_index[0].astype(jnp.int32)
    dst = edge_index[1].astype(jnp.int32)
    pad = EGLUE - E_EDGES
    srcp = jnp.concatenate([src, jnp.zeros((pad,), jnp.int32)])
    dstg = jnp.concatenate([dst, jnp.full((pad,), N_NODES - 1, jnp.int32)])
    dsts = jnp.concatenate([dst, jnp.full((pad,), N_NODES, jnp.int32)])
    gsrc = jnp.stack([srcp, srcp + N_NODES])
    gdst = jnp.stack([dstg, dstg + N_NODES])

    head_of = (jnp.arange(DIM) // CPH)[:, None]
    sel = (head_of == jnp.arange(NH)[None, :]).astype(jnp.float32)
    a_sel_s = sel * att_src.reshape(DIM)[:, None]
    a_sel_d = sel * att_dst.reshape(DIM)[:, None]

    ts0, ts1, td0, td1, h = _project_tables(x, W, a_sel_s, a_sel_d)
    table_src = jnp.concatenate([ts0, ts1], axis=0)
    table_dst = jnp.concatenate([td0, td1], axis=0)

    zeros_init = jnp.zeros((ACC_ROWS, ROW), jnp.float32)
    acc = _edge_aggregate(table_src, table_dst, gsrc, gdst, dsts, zeros_init)

    return _ffn_block(x, h, acc,
                      gat_bias.reshape(1, DIM), ln_g.reshape(1, DIM),
                      ln_b.reshape(1, DIM), W1, b1.reshape(1, 4 * DIM),
                      W2, b2.reshape(1, DIM))


# speedup vs baseline: 1.2679x; 1.2679x over previous
import functools
import math

import jax
import jax.numpy as jnp
from jax import lax
from jax.experimental import pallas as pl
from jax.experimental.pallas import tpu as pltpu
from jax.experimental.pallas import tpu_sc as plsc

N_NODES = 10000
DIM = 256
NH = 8
CPH = 32
E_EDGES = 160000

ROW = 144
DROW = 16
OROW = 136
ACC_ROWS = 10016
CHUNK = 128
BATCH_E = 1024
N_TILES = 16
EPB = 10240
EPAD = EPB * N_TILES
EGLUE = EPAD
BLK = 1000
GRID = N_NODES // BLK
ZSTRIPE = ACC_ROWS // N_TILES


def _a_body(x_ref, w_ref, as_ref, ad_ref, ts0_ref, ts1_ref, td0_ref, td1_ref, h_ref):
    xb = x_ref[...]
    h = jnp.dot(xb, w_ref[...], preferred_element_type=jnp.float32)
    al_s = jnp.dot(h, as_ref[...], preferred_element_type=jnp.float32)
    al_d = jnp.dot(h, ad_ref[...], preferred_element_type=jnp.float32)
    e0 = al_s + al_d
    m_init = jnp.where(e0 > 0.0, e0, 0.2 * e0)
    h_ref[...] = h
    z12 = jnp.zeros((BLK, 12), jnp.float32)
    z8 = jnp.zeros((BLK, 8), jnp.float32)
    ts0_ref[...] = jnp.concatenate([al_s[:, 0:4], z12, h[:, 0:128]], axis=1)
    ts1_ref[...] = jnp.concatenate([al_s[:, 4:8], z12, h[:, 128:256]], axis=1)
    td0_ref[...] = jnp.concatenate([al_d[:, 0:4], m_init[:, 0:4], z8], axis=1)
    td1_ref[...] = jnp.concatenate([al_d[:, 4:8], m_init[:, 4:8], z8], axis=1)


def _project_tables(x, w, a_sel_s, a_sel_d):
    return pl.pallas_call(
        _a_body,
        grid=(GRID,),
        in_specs=[
            pl.BlockSpec((BLK, DIM), lambda i: (i, 0)),
            pl.BlockSpec((DIM, DIM), lambda i: (0, 0)),
            pl.BlockSpec((DIM, NH), lambda i: (0, 0)),
            pl.BlockSpec((DIM, NH), lambda i: (0, 0)),
        ],
        out_specs=[
            pl.BlockSpec((BLK, ROW), lambda i: (i, 0)),
            pl.BlockSpec((BLK, ROW), lambda i: (i, 0)),
            pl.BlockSpec((BLK, DROW), lambda i: (i, 0)),
            pl.BlockSpec((BLK, DROW), lambda i: (i, 0)),
            pl.BlockSpec((BLK, DIM), lambda i: (i, 0)),
        ],
        out_shape=[
            jax.ShapeDtypeStruct((N_NODES, ROW), jnp.float32),
            jax.ShapeDtypeStruct((N_NODES, ROW), jnp.float32),
            jax.ShapeDtypeStruct((N_NODES, DROW), jnp.float32),
            jax.ShapeDtypeStruct((N_NODES, DROW), jnp.float32),
            jax.ShapeDtypeStruct((N_NODES, DIM), jnp.float32),
        ],
    )(x, w, a_sel_s, a_sel_d)


def _b_body(ts_hbm, td_hbm, gsrc_hbm, gdst_hbm, dsts_hbm, zero_hbm, acc_hbm,
            sidx, didx, scat, srows, drows, orows, acc_sh, sem_s, sem_d):
    cid = lax.axis_index("c")
    sid = lax.axis_index("s")

    pltpu.sync_copy(zero_hbm.at[pl.ds(sid * ZSTRIPE, ZSTRIPE)],
                    acc_sh.at[pl.ds(sid * ZSTRIPE, ZSTRIPE)])

    @plsc.parallel_loop(0, CHUNK // 16)
    def zrow(g):
        rows = lax.iota(jnp.int32, 16) + g * 16
        for j in range(4, 8):
            plsc.store_scatter(orows, [rows, jnp.full((16,), j, jnp.int32)],
                               jnp.zeros((16,), jnp.float32))
    plsc.subcore_barrier()

    def batch_body(bi, carry):
        base = sid * EPB + bi * BATCH_E
        pltpu.sync_copy(gsrc_hbm.at[cid, pl.ds(base, BATCH_E)], sidx)
        pltpu.sync_copy(gdst_hbm.at[cid, pl.ds(base, BATCH_E)], didx)
        crow = sid * (EPB // CHUNK) + bi * (BATCH_E // CHUNK)
        pltpu.sync_copy(dsts_hbm.at[pl.ds(crow, BATCH_E // CHUNK)], scat)

        def chunk_body(ck, c1):
            off = ck * CHUNK
            cp1 = pltpu.async_copy(ts_hbm.at[sidx.at[pl.ds(off, CHUNK)]],
                                   srows, sem_s)
            cp2 = pltpu.async_copy(td_hbm.at[didx.at[pl.ds(off, CHUNK)]],
                                   drows, sem_d)
            cp1.wait()
            cp2.wait()

            @plsc.parallel_loop(0, CHUNK // 16)
            def grp(g):
                rows = lax.iota(jnp.int32, 16) + g * 16
                tks = []
                for k in range(4):
                    colk = jnp.full((16,), k, jnp.int32)
                    a_s = plsc.load_gather(srows, [rows, colk])
                    a_d = plsc.load_gather(drows, [rows, colk])
                    m = plsc.load_gather(drows, [rows, jnp.full((16,), 4 + k, jnp.int32)])
                    e = a_s + a_d
                    e = jnp.where(e > 0.0, e, 0.2 * e)
                    tk = jnp.exp(e - m)
                    tks.append(tk)
                    plsc.store_scatter(orows, [rows, colk], tk)
                for cc in range(128):
                    col = jnp.full((16,), 8 + cc, jnp.int32)
                    hv = plsc.load_gather(srows, [rows, jnp.full((16,), 16 + cc, jnp.int32)])
                    plsc.store_scatter(orows, [rows, col], hv * tks[cc // 32])

            pltpu.sync_copy(orows, acc_sh.at[scat.at[ck]], add=True)
            return c1

        lax.fori_loop(0, BATCH_E // CHUNK, chunk_body, 0, unroll=False)
        return carry

    lax.fori_loop(0, EPB // BATCH_E, batch_body, 0, unroll=False)
    plsc.subcore_barrier()
    pltpu.sync_copy(acc_sh.at[pl.ds(sid * ZSTRIPE, ZSTRIPE)],
                    acc_hbm.at[cid, pl.ds(sid * ZSTRIPE, ZSTRIPE)])


def _edge_aggregate(table_src, table_dst, gsrc, gdst, dsts, zeros_init):
    mesh = plsc.VectorSubcoreMesh(core_axis_name="c", subcore_axis_name="s",
                                  num_cores=2, num_subcores=N_TILES)
    return pl.kernel(
        _b_body,
        out_type=jax.ShapeDtypeStruct((2, ACC_ROWS, OROW), jnp.float32),
        mesh=mesh,
        scratch_types=(
            [pltpu.VMEM((BATCH_E,), jnp.int32)] * 2
            + [pltpu.VMEM((BATCH_E // CHUNK, CHUNK), jnp.int32)]
            + [pltpu.VMEM((CHUNK, ROW), jnp.float32)]
            + [pltpu.VMEM((CHUNK, DROW), jnp.float32)]
            + [pltpu.VMEM((CHUNK, OROW), jnp.float32)]
            + [pltpu.VMEM_SHARED((ACC_ROWS, OROW), jnp.float32)]
            + [pltpu.SemaphoreType.DMA] * 2
        ),
        compiler_params=pltpu.CompilerParams(use_tc_tiling_on_sc=False,
                                             needs_layout_passes=False),
    )(table_src, table_dst, gsrc, gdst, dsts, zeros_init)


def _c_body(x_ref, h_ref, a0_ref, a1_ref, gb_ref, lg_ref, lb_ref,
            w1_ref, b1_ref, w2_ref, b2_ref, o_ref):
    xb = x_ref[...]
    hb = h_ref[...]
    a0 = a0_ref[...]
    a1 = a1_ref[...]
    msg = jnp.concatenate([a0[:, 8:OROW], a1[:, 8:OROW]], axis=1)
    den = 1.0 + jnp.concatenate([a0[:, 0:4], a1[:, 0:4]], axis=1)
    rec = 1.0 / den
    head_of = lax.broadcasted_iota(jnp.int32, (NH, DIM), 1) // CPH
    hid = lax.broadcasted_iota(jnp.int32, (NH, DIM), 0)
    sel = (head_of == hid).astype(jnp.float32)
    recb = jnp.dot(rec, sel, preferred_element_type=jnp.float32)
    gat = (hb + msg) * recb + gb_ref[...]

    lg = lg_ref[...]
    lb = lb_ref[...]

    def ln(v):
        mu = jnp.mean(v, axis=-1, keepdims=True)
        var = jnp.mean((v - mu) ** 2, axis=-1, keepdims=True)
        return (v - mu) * lax.rsqrt(var + 1e-5) * lg + lb

    h1 = ln(xb + gat)
    aa = jnp.dot(h1, w1_ref[...], preferred_element_type=jnp.float32) + b1_ref[...]
    gg = 0.5 * aa * (1.0 + lax.erf(aa * (1.0 / math.sqrt(2.0))))
    ff = jnp.dot(gg, w2_ref[...], preferred_element_type=jnp.float32) + b2_ref[...]
    o_ref[...] = ln(h1 + ff)


def _ffn_block(x, h, acc, gat_bias, ln_g, ln_b, w1, b1, w2, b2):
    full = lambda shape: pl.BlockSpec(shape, lambda i: tuple(0 for _ in shape))
    return pl.pallas_call(
        _c_body,
        grid=(GRID,),
        in_specs=[
            pl.BlockSpec((BLK, DIM), lambda i: (i, 0)),
            pl.BlockSpec((BLK, DIM), lambda i: (i, 0)),
            pl.BlockSpec((None, BLK, OROW), lambda i: (0, i, 0)),
            pl.BlockSpec((None, BLK, OROW), lambda i: (1, i, 0)),
            full((1, DIM)),
            full((1, DIM)),
            full((1, DIM)),
            full((DIM, 4 * DIM)),
            full((1, 4 * DIM)),
            full((4 * DIM, DIM)),
            full((1, DIM)),
        ],
        out_specs=pl.BlockSpec((BLK, DIM), lambda i: (i, 0)),
        out_shape=jax.ShapeDtypeStruct((N_NODES, DIM), jnp.float32),
    )(x, h, acc, acc, gat_bias, ln_g, ln_b, w1, b1, w2, b2)


def kernel(x, edge_index, W, att_src, att_dst, gat_bias, ln_g, ln_b, W1, b1, W2, b2):
    src = edge_index[0].astype(jnp.int32)
    dst = edge_index[1].astype(jnp.int32)
    pad = EGLUE - E_EDGES
    srcp = jnp.concatenate([src, jnp.zeros((pad,), jnp.int32)])
    dstg = jnp.concatenate([dst, jnp.full((pad,), N_NODES - 1, jnp.int32)])
    dsts = jnp.concatenate([dst, jnp.full((pad,), N_NODES, jnp.int32)])
    dsts = dsts.reshape(EPAD // CHUNK, CHUNK)
    gsrc = jnp.stack([srcp, srcp + N_NODES])
    gdst = jnp.stack([dstg, dstg + N_NODES])

    head_of = (jnp.arange(DIM) // CPH)[:, None]
    sel = (head_of == jnp.arange(NH)[None, :]).astype(jnp.float32)
    a_sel_s = sel * att_src.reshape(DIM)[:, None]
    a_sel_d = sel * att_dst.reshape(DIM)[:, None]

    ts0, ts1, td0, td1, h = _project_tables(x, W, a_sel_s, a_sel_d)
    table_src = jnp.concatenate([ts0, ts1], axis=0)
    table_dst = jnp.concatenate([td0, td1], axis=0)

    zeros_init = jnp.zeros((ACC_ROWS, OROW), jnp.float32)
    acc = _edge_aggregate(table_src, table_dst, gsrc, gdst, dsts, zeros_init)

    return _ffn_block(x, h, acc,
                      gat_bias.reshape(1, DIM), ln_g.reshape(1, DIM),
                      ln_b.reshape(1, DIM), W1, b1.reshape(1, 4 * DIM),
                      W2, b2.reshape(1, DIM))


# speedup vs baseline: 1.7786x; 1.4027x over previous
import functools
import math

import jax
import jax.numpy as jnp
from jax import lax
from jax.experimental import pallas as pl
from jax.experimental.pallas import tpu as pltpu
from jax.experimental.pallas import tpu_sc as plsc

N_NODES = 10000
DIM = 256
NH = 8
CPH = 32
E_EDGES = 160000

ROW = 80
DROW = 16
OROW = 136
ACC_ROWS = 10016
CHUNK = 128
BATCH_E = 1024
N_TILES = 16
EPB = 10240
EPAD = EPB * N_TILES
EGLUE = EPAD
BLK = 1000
GRID = N_NODES // BLK
ZSTRIPE = ACC_ROWS // N_TILES


def _a_body(x_ref, w_ref, as_ref, ad_ref, ts0_ref, ts1_ref, td0_ref, td1_ref, h_ref):
    xb = x_ref[...]
    h = jnp.dot(xb, w_ref[...], preferred_element_type=jnp.float32)
    al_s = jnp.dot(h, as_ref[...], preferred_element_type=jnp.float32)
    al_d = jnp.dot(h, ad_ref[...], preferred_element_type=jnp.float32)
    e0 = al_s + al_d
    m_init = jnp.where(e0 > 0.0, e0, 0.2 * e0)
    h_ref[...] = h

    def pack_half(hh):
        rt = lambda v: lax.bitcast_convert_type(
            v.astype(jnp.bfloat16).astype(jnp.float32), jnp.int32)
        lo = rt(hh[:, 0:64])
        hi = rt(hh[:, 64:128])
        word = jax.lax.bitwise_or(
            jax.lax.shift_right_logical(lo, 16),
            jax.lax.bitwise_and(hi, jnp.int32(-65536)))
        return lax.bitcast_convert_type(word, jnp.float32)

    z12 = jnp.zeros((BLK, 12), jnp.float32)
    z8 = jnp.zeros((BLK, 8), jnp.float32)
    ts0_ref[...] = jnp.concatenate([al_s[:, 0:4], z12, pack_half(h[:, 0:128])], axis=1)
    ts1_ref[...] = jnp.concatenate([al_s[:, 4:8], z12, pack_half(h[:, 128:256])], axis=1)
    td0_ref[...] = jnp.concatenate([al_d[:, 0:4], m_init[:, 0:4], z8], axis=1)
    td1_ref[...] = jnp.concatenate([al_d[:, 4:8], m_init[:, 4:8], z8], axis=1)


def _project_tables(x, w, a_sel_s, a_sel_d):
    return pl.pallas_call(
        _a_body,
        grid=(GRID,),
        in_specs=[
            pl.BlockSpec((BLK, DIM), lambda i: (i, 0)),
            pl.BlockSpec((DIM, DIM), lambda i: (0, 0)),
            pl.BlockSpec((DIM, NH), lambda i: (0, 0)),
            pl.BlockSpec((DIM, NH), lambda i: (0, 0)),
        ],
        out_specs=[
            pl.BlockSpec((BLK, ROW), lambda i: (i, 0)),
            pl.BlockSpec((BLK, ROW), lambda i: (i, 0)),
            pl.BlockSpec((BLK, DROW), lambda i: (i, 0)),
            pl.BlockSpec((BLK, DROW), lambda i: (i, 0)),
            pl.BlockSpec((BLK, DIM), lambda i: (i, 0)),
        ],
        out_shape=[
            jax.ShapeDtypeStruct((N_NODES, ROW), jnp.float32),
            jax.ShapeDtypeStruct((N_NODES, ROW), jnp.float32),
            jax.ShapeDtypeStruct((N_NODES, DROW), jnp.float32),
            jax.ShapeDtypeStruct((N_NODES, DROW), jnp.float32),
            jax.ShapeDtypeStruct((N_NODES, DIM), jnp.float32),
        ],
    )(x, w, a_sel_s, a_sel_d)


def _b_body(ts_hbm, td_hbm, gsrc_hbm, gdst_hbm, dsts_hbm, zero_hbm, acc_hbm,
            sidx, didx, scat, srows0, srows1, drows0, drows1, orows, acc_sh,
            sem_s0, sem_s1, sem_d0, sem_d1):
    cid = lax.axis_index("c")
    sid = lax.axis_index("s")
    bufs = [(srows0, drows0, sem_s0, sem_d0), (srows1, drows1, sem_s1, sem_d1)]

    pltpu.sync_copy(zero_hbm.at[pl.ds(sid * ZSTRIPE, ZSTRIPE)],
                    acc_sh.at[pl.ds(sid * ZSTRIPE, ZSTRIPE)])

    @plsc.parallel_loop(0, CHUNK // 16)
    def zrow(g):
        rows = lax.iota(jnp.int32, 16) + g * 16
        for j in range(4, 8):
            plsc.store_scatter(orows, [rows, jnp.full((16,), j, jnp.int32)],
                               jnp.zeros((16,), jnp.float32))
    plsc.subcore_barrier()

    def prefetch(off, b):
        srows, drows, sem_s, sem_d = b
        pltpu.async_copy(ts_hbm.at[sidx.at[pl.ds(off, CHUNK)]], srows, sem_s)
        pltpu.async_copy(td_hbm.at[didx.at[pl.ds(off, CHUNK)]], drows, sem_d)

    def gwait(b):
        srows, drows, sem_s, sem_d = b
        pltpu.make_async_copy(ts_hbm.at[sidx.at[pl.ds(0, CHUNK)]], srows,
                              sem_s).wait()
        pltpu.make_async_copy(td_hbm.at[didx.at[pl.ds(0, CHUNK)]], drows,
                              sem_d).wait()

    def work(ck, b):
        srows, drows, _, _ = b

        @plsc.parallel_loop(0, CHUNK // 16)
        def grp(g):
            rows = lax.iota(jnp.int32, 16) + g * 16
            tks = []
            for k in range(4):
                colk = jnp.full((16,), k, jnp.int32)
                a_s = plsc.load_gather(srows, [rows, colk])
                a_d = plsc.load_gather(drows, [rows, colk])
                m = plsc.load_gather(drows, [rows, jnp.full((16,), 4 + k, jnp.int32)])
                e = a_s + a_d
                e = jnp.where(e > 0.0, e, 0.2 * e)
                tk = jnp.exp(e - m)
                tks.append(tk)
                plsc.store_scatter(orows, [rows, colk], tk)
            for j in range(64):
                w = plsc.load_gather(srows, [rows, jnp.full((16,), 16 + j, jnp.int32)])
                hlo, hhi = plsc.unpack(plsc.bitcast(w, jnp.bfloat16),
                                       format=plsc.PackFormat.INTERLEAVED)
                plsc.store_scatter(orows, [rows, jnp.full((16,), 8 + j, jnp.int32)],
                                   hlo * tks[j // 32])
                plsc.store_scatter(orows, [rows, jnp.full((16,), 72 + j, jnp.int32)],
                                   hhi * tks[2 + j // 32])

        pltpu.sync_copy(orows, acc_sh.at[scat.at[ck]], add=True)

    ncpb = BATCH_E // CHUNK

    def batch_body(bi, carry):
        base = sid * EPB + bi * BATCH_E
        pltpu.sync_copy(gsrc_hbm.at[cid, pl.ds(base, BATCH_E)], sidx)
        pltpu.sync_copy(gdst_hbm.at[cid, pl.ds(base, BATCH_E)], didx)
        crow = sid * (EPB // CHUNK) + bi * ncpb
        pltpu.sync_copy(dsts_hbm.at[pl.ds(crow, ncpb)], scat)
        prefetch(0, bufs[0])

        def pair(k2, c1):
            ck0 = k2 * 2
            prefetch((ck0 + 1) * CHUNK, bufs[1])
            gwait(bufs[0])
            work(ck0, bufs[0])
            off2 = jnp.minimum((ck0 + 2) * CHUNK, (ncpb - 1) * CHUNK)
            prefetch(off2, bufs[0])
            gwait(bufs[1])
            work(ck0 + 1, bufs[1])
            return c1

        lax.fori_loop(0, ncpb // 2, pair, 0, unroll=False)
        gwait(bufs[0])
        return carry

    lax.fori_loop(0, EPB // BATCH_E, batch_body, 0, unroll=False)
    plsc.subcore_barrier()
    pltpu.sync_copy(acc_sh.at[pl.ds(sid * ZSTRIPE, ZSTRIPE)],
                    acc_hbm.at[cid, pl.ds(sid * ZSTRIPE, ZSTRIPE)])


def _edge_aggregate(table_src, table_dst, gsrc, gdst, dsts, zeros_init):
    mesh = plsc.VectorSubcoreMesh(core_axis_name="c", subcore_axis_name="s",
                                  num_cores=2, num_subcores=N_TILES)
    return pl.kernel(
        _b_body,
        out_type=jax.ShapeDtypeStruct((2, ACC_ROWS, OROW), jnp.float32),
        mesh=mesh,
        scratch_types=(
            [pltpu.VMEM((BATCH_E,), jnp.int32)] * 2
            + [pltpu.VMEM((BATCH_E // CHUNK, CHUNK), jnp.int32)]
            + [pltpu.VMEM((CHUNK, ROW), jnp.float32)] * 2
            + [pltpu.VMEM((CHUNK, DROW), jnp.float32)] * 2
            + [pltpu.VMEM((CHUNK, OROW), jnp.float32)]
            + [pltpu.VMEM_SHARED((ACC_ROWS, OROW), jnp.float32)]
            + [pltpu.SemaphoreType.DMA] * 4
        ),
        compiler_params=pltpu.CompilerParams(use_tc_tiling_on_sc=False,
                                             needs_layout_passes=False),
    )(table_src, table_dst, gsrc, gdst, dsts, zeros_init)


def _c_body(x_ref, h_ref, a0_ref, a1_ref, gb_ref, lg_ref, lb_ref,
            w1_ref, b1_ref, w2_ref, b2_ref, o_ref):
    xb = x_ref[...]
    hb = h_ref[...]
    a0 = a0_ref[...]
    a1 = a1_ref[...]
    msg = jnp.concatenate([a0[:, 8:OROW], a1[:, 8:OROW]], axis=1)
    den = 1.0 + jnp.concatenate([a0[:, 0:4], a1[:, 0:4]], axis=1)
    rec = 1.0 / den
    head_of = lax.broadcasted_iota(jnp.int32, (NH, DIM), 1) // CPH
    hid = lax.broadcasted_iota(jnp.int32, (NH, DIM), 0)
    sel = (head_of == hid).astype(jnp.float32)
    recb = jnp.dot(rec, sel, preferred_element_type=jnp.float32)
    gat = (hb + msg) * recb + gb_ref[...]

    lg = lg_ref[...]
    lb = lb_ref[...]

    def ln(v):
        mu = jnp.mean(v, axis=-1, keepdims=True)
        var = jnp.mean((v - mu) ** 2, axis=-1, keepdims=True)
        return (v - mu) * lax.rsqrt(var + 1e-5) * lg + lb

    h1 = ln(xb + gat)
    aa = jnp.dot(h1, w1_ref[...], preferred_element_type=jnp.float32) + b1_ref[...]
    gg = 0.5 * aa * (1.0 + lax.erf(aa * (1.0 / math.sqrt(2.0))))
    ff = jnp.dot(gg, w2_ref[...], preferred_element_type=jnp.float32) + b2_ref[...]
    o_ref[...] = ln(h1 + ff)


def _ffn_block(x, h, acc, gat_bias, ln_g, ln_b, w1, b1, w2, b2):
    full = lambda shape: pl.BlockSpec(shape, lambda i: tuple(0 for _ in shape))
    return pl.pallas_call(
        _c_body,
        grid=(GRID,),
        in_specs=[
            pl.BlockSpec((BLK, DIM), lambda i: (i, 0)),
            pl.BlockSpec((BLK, DIM), lambda i: (i, 0)),
            pl.BlockSpec((None, BLK, OROW), lambda i: (0, i, 0)),
            pl.BlockSpec((None, BLK, OROW), lambda i: (1, i, 0)),
            full((1, DIM)),
            full((1, DIM)),
            full((1, DIM)),
            full((DIM, 4 * DIM)),
            full((1, 4 * DIM)),
            full((4 * DIM, DIM)),
            full((1, DIM)),
        ],
        out_specs=pl.BlockSpec((BLK, DIM), lambda i: (i, 0)),
        out_shape=jax.ShapeDtypeStruct((N_NODES, DIM), jnp.float32),
    )(x, h, acc, acc, gat_bias, ln_g, ln_b, w1, b1, w2, b2)


def kernel(x, edge_index, W, att_src, att_dst, gat_bias, ln_g, ln_b, W1, b1, W2, b2):
    src = edge_index[0].astype(jnp.int32)
    dst = edge_index[1].astype(jnp.int32)
    pad = EGLUE - E_EDGES
    srcp = jnp.concatenate([src, jnp.zeros((pad,), jnp.int32)])
    dstg = jnp.concatenate([dst, jnp.full((pad,), N_NODES - 1, jnp.int32)])
    dsts = jnp.concatenate([dst, jnp.full((pad,), N_NODES, jnp.int32)])
    dsts = dsts.reshape(EPAD // CHUNK, CHUNK)
    gsrc = jnp.stack([srcp, srcp + N_NODES])
    gdst = jnp.stack([dstg, dstg + N_NODES])

    head_of = (jnp.arange(DIM) // CPH)[:, None]
    sel = (head_of == jnp.arange(NH)[None, :]).astype(jnp.float32)
    a_sel_s = sel * att_src.reshape(DIM)[:, None]
    a_sel_d = sel * att_dst.reshape(DIM)[:, None]

    ts0, ts1, td0, td1, h = _project_tables(x, W, a_sel_s, a_sel_d)
    table_src = jnp.concatenate([ts0, ts1], axis=0)
    table_dst = jnp.concatenate([td0, td1], axis=0)

    zeros_init = jnp.zeros((ACC_ROWS, OROW), jnp.float32)
    acc = _edge_aggregate(table_src, table_dst, gsrc, gdst, dsts, zeros_init)

    return _ffn_block(x, h, acc,
                      gat_bias.reshape(1, DIM), ln_g.reshape(1, DIM),
                      ln_b.reshape(1, DIM), W1, b1.reshape(1, 4 * DIM),
                      W2, b2.reshape(1, DIM))
